# baseline (speedup 1.0000x reference)
import jax
import jax.numpy as jnp
from jax.experimental import pallas as pl

_NUM_CLASSES = 10
_NUM_QUERY = 900
_NUM_PROPAGATED = 256
_G = 5
_BBOX_NOISE_SCALE = 0.4
_NOISE_THRESH = 0.5
_ONES32 = 0x01010101


def _body(gt_ref, lab_ref, rp_ref, rnd_ref, cor_ref,
          padded_ref, mask_ref, labels_ref, bboxes_ref):
    B, N = lab_ref.shape
    G = _G
    pad = G * N
    tgt = mask_ref.shape[0]
    N4 = N // 4
    pad4 = pad // 4

    gt = gt_ref[...]
    centers = gt[:, :, 0:3]
    diff = gt[:, :, 3:6] * 0.5
    kidx = jax.lax.broadcasted_iota(jnp.int32, (1, 1, 3), 2)
    lo = jnp.where(kidx == 2, -8.0, -65.0).astype(jnp.float32)
    rng = jnp.where(kidx == 2, 16.0, 130.0).astype(jnp.float32)
    lab = lab_ref[...]
    for g in range(G):
        noisy = centers + rnd_ref[g] * diff * _BBOX_NOISE_SCALE
        noisy = jnp.clip((noisy - lo) / rng, 0.0, 1.0)
        padded_ref[:, g * N:(g + 1) * N, :] = noisy
        labels_ref[g] = jnp.where(cor_ref[g] != 0, _NUM_CLASSES, lab)
        bboxes_ref[g] = gt
    padded_ref[:, pad:, :] = jnp.broadcast_to(rp_ref[...][None],
                                              (B, _NUM_QUERY, 3))

    mask_ref[:, pad4:] = jnp.zeros((tgt, mask_ref.shape[1] - pad4), jnp.int32)
    mask_ref[:, :pad4] = jnp.full((tgt, pad4), _ONES32, jnp.int32)
    for g in range(G):
        mask_ref[g * N:(g + 1) * N, g * N4:(g + 1) * N4] = (
            jnp.zeros((N, N4), jnp.int32))


def kernel(batch_size, reference_points, gt_bboxes_3d, gt_labels):
    B, N = gt_labels.shape
    G = _G
    pad = G * N
    total_q = pad + _NUM_QUERY
    tgt = total_q + _NUM_PROPAGATED

    with jax.ensure_compile_time_eval():
        nk = jax.random.key(12345)
        rand_prob = jax.random.uniform(nk, (G * B * N, 3), dtype=jnp.float32)
        rand_prob = rand_prob * 2.0 - 1.0
        corrupt = (jnp.linalg.norm(rand_prob, axis=1) > _NOISE_THRESH)
        rnd = rand_prob.reshape(G, B, N, 3)
        cor = corrupt.astype(jnp.int32).reshape(G, B, N)

    out_shape = (
        jax.ShapeDtypeStruct((B, total_q, 3), jnp.float32),
        jax.ShapeDtypeStruct((tgt, tgt // 4), jnp.int32),
        jax.ShapeDtypeStruct((G, B, N), jnp.int32),
        jax.ShapeDtypeStruct((G, B, N, 9), jnp.float32),
    )
    padded, mask32, labels, bboxes = pl.pallas_call(
        _body,
        out_shape=out_shape,
    )(gt_bboxes_3d, gt_labels, reference_points, rnd, cor)
    mask_bytes = jax.lax.bitcast_convert_type(mask32, jnp.uint8)
    mask = mask_bytes.reshape(tgt, tgt).astype(jnp.bool_)
    return (padded, mask,
            labels.reshape(G * B * N), bboxes.reshape(G * B * N, 9))


# speedup vs baseline: 2.6804x; 2.6804x over previous
import jax
import jax.numpy as jnp
from jax.experimental import pallas as pl

_NUM_CLASSES = 10
_NUM_QUERY = 900
_NUM_PROPAGATED = 256
_G = 5
_BBOX_NOISE_SCALE = 0.4
_NOISE_THRESH = 0.5


def _body(gt_ref, lab_ref, rp_ref, rnd_ref, cor_ref,
          padded_ref, mask_ref, labels_ref, bboxes_ref):
    B, N = lab_ref.shape
    G = _G
    pad = G * N
    tgt = mask_ref.shape[0]

    gt = gt_ref[...]
    centers = gt[:, :, 0:3]
    diff = gt[:, :, 3:6] * 0.5
    kidx = jax.lax.broadcasted_iota(jnp.int32, (1, 1, 3), 2)
    lo = jnp.where(kidx == 2, -8.0, -65.0).astype(jnp.float32)
    rng = jnp.where(kidx == 2, 16.0, 130.0).astype(jnp.float32)
    lab = lab_ref[...]
    for g in range(G):
        noisy = centers + rnd_ref[g] * diff * _BBOX_NOISE_SCALE
        noisy = jnp.clip((noisy - lo) / rng, 0.0, 1.0)
        padded_ref[:, g * N:(g + 1) * N, :] = noisy
        labels_ref[g] = jnp.where(cor_ref[g] != 0, _NUM_CLASSES, lab)
        bboxes_ref[g] = gt
    padded_ref[:, pad:, :] = jnp.broadcast_to(rp_ref[...][None],
                                              (B, _NUM_QUERY, 3))

    mask_ref[:, pad:] = jnp.zeros((tgt, tgt - pad), jnp.int8)
    mask_ref[:, :pad] = jnp.ones((tgt, pad), jnp.int8)
    for g in range(G):
        mask_ref[g * N:(g + 1) * N, g * N:(g + 1) * N] = (
            jnp.zeros((N, N), jnp.int8))


def kernel(batch_size, reference_points, gt_bboxes_3d, gt_labels):
    B, N = gt_labels.shape
    G = _G
    pad = G * N
    total_q = pad + _NUM_QUERY
    tgt = total_q + _NUM_PROPAGATED

    with jax.ensure_compile_time_eval():
        nk = jax.random.key(12345)
        rand_prob = jax.random.uniform(nk, (G * B * N, 3), dtype=jnp.float32)
        rand_prob = rand_prob * 2.0 - 1.0
        corrupt = (jnp.linalg.norm(rand_prob, axis=1) > _NOISE_THRESH)
        rnd = rand_prob.reshape(G, B, N, 3)
        cor = corrupt.astype(jnp.int32).reshape(G, B, N)

    out_shape = (
        jax.ShapeDtypeStruct((B, total_q, 3), jnp.float32),
        jax.ShapeDtypeStruct((tgt, tgt), jnp.int8),
        jax.ShapeDtypeStruct((G, B, N), jnp.int32),
        jax.ShapeDtypeStruct((G, B, N, 9), jnp.float32),
    )
    padded, mask, labels, bboxes = pl.pallas_call(
        _body,
        out_shape=out_shape,
    )(gt_bboxes_3d, gt_labels, reference_points, rnd, cor)
    return (padded, mask.astype(jnp.bool_),
            labels.reshape(G * B * N), bboxes.reshape(G * B * N, 9))


# speedup vs baseline: 2.9461x; 1.0991x over previous
import jax
import jax.numpy as jnp
from jax.experimental import pallas as pl

_NUM_CLASSES = 10
_NUM_QUERY = 900
_NUM_PROPAGATED = 256
_G = 5
_BBOX_NOISE_SCALE = 0.4
_NOISE_THRESH = 0.5


def _body(gt_ref, lab_ref, rp_ref, rnd_ref, cor_ref,
          padded_ref, labels_ref, bboxes_ref):
    B, N = lab_ref.shape
    G = _G
    pad = G * N

    gt = gt_ref[...]
    centers = gt[:, :, 0:3]
    diff = gt[:, :, 3:6] * 0.5
    kidx = jax.lax.broadcasted_iota(jnp.int32, (1, 1, 3), 2)
    lo = jnp.where(kidx == 2, -8.0, -65.0).astype(jnp.float32)
    rng = jnp.where(kidx == 2, 16.0, 130.0).astype(jnp.float32)
    lab = lab_ref[...]
    for g in range(G):
        noisy = centers + rnd_ref[g] * diff * _BBOX_NOISE_SCALE
        noisy = jnp.clip((noisy - lo) / rng, 0.0, 1.0)
        padded_ref[:, g * N:(g + 1) * N, :] = noisy
        labels_ref[g] = jnp.where(cor_ref[g] != 0, _NUM_CLASSES, lab)
        bboxes_ref[g] = gt
    padded_ref[:, pad:, :] = jnp.broadcast_to(rp_ref[...][None],
                                              (B, _NUM_QUERY, 3))


def kernel(batch_size, reference_points, gt_bboxes_3d, gt_labels):
    B, N = gt_labels.shape
    G = _G
    pad = G * N
    total_q = pad + _NUM_QUERY
    tgt = total_q + _NUM_PROPAGATED

    with jax.ensure_compile_time_eval():
        nk = jax.random.key(12345)
        rand_prob = jax.random.uniform(nk, (G * B * N, 3), dtype=jnp.float32)
        rand_prob = rand_prob * 2.0 - 1.0
        corrupt = (jnp.linalg.norm(rand_prob, axis=1) > _NOISE_THRESH)
        rnd = rand_prob.reshape(G, B, N, 3)
        cor = corrupt.astype(jnp.int32).reshape(G, B, N)

    out_shape = (
        jax.ShapeDtypeStruct((B, total_q, 3), jnp.float32),
        jax.ShapeDtypeStruct((G, B, N), jnp.int32),
        jax.ShapeDtypeStruct((G, B, N, 9), jnp.float32),
    )
    padded, labels, bboxes = pl.pallas_call(
        _body,
        out_shape=out_shape,
    )(gt_bboxes_3d, gt_labels, reference_points, rnd, cor)
    row = jax.lax.broadcasted_iota(jnp.int32, (tgt, tgt), 0)
    col = jax.lax.broadcasted_iota(jnp.int32, (tgt, tgt), 1)
    mask = (col < pad) & ((row // N) != (col // N))
    return (padded, mask,
            labels.reshape(G * B * N), bboxes.reshape(G * B * N, 9))
